# Initial kernel scaffold; baseline (speedup 1.0000x reference)
import jax
import jax.numpy as jnp
from jax.experimental import pallas as pl


def kernel(x, row_idx, col_idx, values, bias):
    raise NotImplementedError("write your pallas kernel here")


# speedup vs baseline: 7.3822x; 7.3822x over previous
import functools

import jax
import jax.numpy as jnp
from jax import lax
from jax.experimental import pallas as pl
from jax.experimental.pallas import tpu as pltpu
from jax.experimental.pallas import tpu_sc as plsc

_NC = 2
_NS = 16
_NW = _NC * _NS
_L = 16

_CHUNK = 2048


def _sc_sparse_linear(nnz_pad, batch, in_features, out_features):
    b_per_w = batch // _NW
    n_chunks = nnz_pad // _CHUNK
    assert n_chunks % 2 == 0
    groups = _CHUNK // _L

    mesh = plsc.VectorSubcoreMesh(
        core_axis_name="c", subcore_axis_name="s", num_cores=_NC,
        num_subcores=_NS)

    @functools.partial(
        pl.kernel,
        mesh=mesh,
        out_type=jax.ShapeDtypeStruct((batch * out_features,), jnp.float32),
        scratch_types=[
            pltpu.VMEM((b_per_w * in_features,), jnp.float32),
            pltpu.VMEM((b_per_w * out_features,), jnp.float32),
            pltpu.VMEM((2 * _CHUNK,), jnp.int32),
            pltpu.VMEM((2 * _CHUNK,), jnp.int32),
            pltpu.SemaphoreType.DMA,
            pltpu.SemaphoreType.DMA,
        ],
        compiler_params=pltpu.CompilerParams(needs_layout_passes=False),
    )
    def kern(x_hbm, coo_hbm, bias_hbm, out_hbm, x_v, o_v, s0_v, s1_v,
             sem0, sem1):
        wid = lax.axis_index("s") * _NC + lax.axis_index("c")
        xw = b_per_w * in_features
        ow = b_per_w * out_features
        pltpu.sync_copy(x_hbm.at[pl.ds(wid * xw, xw)], x_v)
        for j in range(b_per_w):
            pltpu.sync_copy(bias_hbm,
                            o_v.at[pl.ds(j * out_features, out_features)])

        def start(c, buf, sem):
            pltpu.async_copy(coo_hbm.at[pl.ds(c * 2 * _CHUNK, 2 * _CHUNK)],
                             buf, sem)

        def drain(buf, sem):
            pltpu.make_async_copy(
                coo_hbm.at[pl.ds(0, 2 * _CHUNK)], buf, sem).wait()

        def process(buf):
            @plsc.parallel_loop(0, groups, 1, unroll=4)
            def g_body(g):
                vp = buf[pl.ds(g * _L, _L)]
                vc = vp & jnp.int32(0xFFFF)
                vr = lax.shift_right_logical(vp, jnp.int32(16))
                vv = plsc.bitcast(buf[pl.ds(_CHUNK + g * _L, _L)],
                                  jnp.float32)
                xs = [plsc.load_gather(
                          x_v.at[pl.ds(j * in_features, in_features)], [vc])
                      for j in range(b_per_w)]
                ps = [xs[j] * vv for j in range(b_per_w)]
                for j in range(b_per_w):
                    plsc.addupdate_scatter(
                        o_v.at[pl.ds(j * out_features, out_features)],
                        [vr], ps[j])

        start(0, s0_v, sem0)

        def pair_body(i, carry):
            drain(s0_v, sem0)
            start(2 * i + 1, s1_v, sem1)
            process(s0_v)
            drain(s1_v, sem1)
            start(2 * i + 2, s0_v, sem0)
            process(s1_v)
            return carry

        lax.fori_loop(0, n_chunks // 2, pair_body, 0)
        drain(s0_v, sem0)

        pltpu.sync_copy(o_v, out_hbm.at[pl.ds(wid * ow, ow)])

    return kern


def kernel(x, row_idx, col_idx, values, bias):
    input_shape = x.shape
    in_features = x.shape[-1]
    x_flat = x.reshape(-1, in_features)
    batch = x_flat.shape[0]
    out_features = bias.shape[0]
    nnz = values.shape[0]

    n_chunks = ((nnz + _CHUNK - 1) // _CHUNK + 1) // 2 * 2
    nnz_pad = n_chunks * _CHUNK
    pad = nnz_pad - nnz
    zpad = jnp.zeros((pad,), jnp.int32)
    packed = jnp.concatenate([col_idx | (row_idx << 16), zpad])
    val_p = jnp.concatenate(
        [lax.bitcast_convert_type(values, jnp.int32), zpad])
    coo = jnp.stack([packed.reshape(n_chunks, _CHUNK),
                     val_p.reshape(n_chunks, _CHUNK)], axis=1).reshape(-1)
    coo = jnp.concatenate([coo, jnp.zeros((2 * _CHUNK,), jnp.int32)])

    kern = _sc_sparse_linear(nnz_pad, batch, in_features, out_features)
    out = kern(x_flat.reshape(-1), coo, bias)
    return out.reshape(input_shape[:-1] + (out_features,))
